# baseline (speedup 1.0000x reference)
import functools

import jax
import jax.numpy as jnp
from jax import lax
from jax.experimental import pallas as pl
from jax.experimental.pallas import tpu as pltpu
from jax.experimental.pallas import tpu_sc as plsc

E = 160000
T = 480000
H = 128
F = 64
NR = 6
NSR = 42

NCORE = 2
NSUB = 16

C = 20000
KCH = 4
EP = NCORE * KCH * C
TS = T // NSUB
TW = T // (NCORE * NSUB)
GB = 600
MB = 3000
NMB = TS // MB
AROWS = 24
NGRP = 32
ZROWS = (C + 128) // NSUB
DRAIN = C // NSUB

_f32 = jnp.float32


def _swish(x):
    return x * jax.nn.sigmoid(x)


def _edges_body(x1_ref, rbf0_ref, wji_ref, bji_ref, wkj_ref, bkj_ref,
                wr1_ref, wr2_ref, wd_ref, xji_ref, xkj_ref):
    x1 = x1_ref[...]
    a = jnp.dot(x1, wji_ref[...], preferred_element_type=_f32) + bji_ref[...]
    xji_ref[...] = _swish(a)
    b = jnp.dot(x1, wkj_ref[...], preferred_element_type=_f32) + bkj_ref[...]
    r1 = jnp.dot(rbf0_ref[...], wr1_ref[...], preferred_element_type=_f32)
    rbf = jnp.dot(r1, wr2_ref[...], preferred_element_type=_f32)
    c = _swish(b) * rbf
    d = jnp.dot(c, wd_ref[...], preferred_element_type=_f32)
    xkj_ref[...] = _swish(d)


def _edges(x1, rbf0, w_ji, b_ji, w_kj, b_kj, w_rbf1, w_rbf2, w_down):
    be = 2000
    grid = (E // be,)
    full = lambda shape: pl.BlockSpec(shape, lambda i: (0, 0))
    return pl.pallas_call(
        _edges_body,
        grid=grid,
        in_specs=[
            pl.BlockSpec((be, H), lambda i: (i, 0)),
            pl.BlockSpec((be, NR), lambda i: (i, 0)),
            full((H, H)), full((1, H)), full((H, H)), full((1, H)),
            full((NR, 8)), full((8, H)), full((H, F)),
        ],
        out_specs=[
            pl.BlockSpec((be, H), lambda i: (i, 0)),
            pl.BlockSpec((be, F), lambda i: (i, 0)),
        ],
        out_shape=[
            jax.ShapeDtypeStruct((E, H), _f32),
            jax.ShapeDtypeStruct((E, F), _f32),
        ],
        compiler_params=pltpu.CompilerParams(
            dimension_semantics=("parallel",)),
    )(x1, rbf0, w_ji, b_ji, w_kj, b_kj, w_rbf1, w_rbf2, w_down)


def _prod_body(sbf_ref, g_ref, w1_ref, w2_ref, out_ref):
    s1 = jnp.dot(sbf_ref[...], w1_ref[...], preferred_element_type=_f32)
    s = jnp.dot(s1, w2_ref[...], preferred_element_type=_f32)
    out_ref[...] = s * g_ref[...]


def _prod(sbf, gathered, w_sbf1, w_sbf2):
    bt = 4000
    grid = (T // bt,)
    full = lambda shape: pl.BlockSpec(shape, lambda i: (0, 0))
    return pl.pallas_call(
        _prod_body,
        grid=grid,
        in_specs=[
            pl.BlockSpec((bt, NSR), lambda i: (i, 0)),
            pl.BlockSpec((bt, F), lambda i: (i, 0)),
            full((NSR, F)), full((F, F)),
        ],
        out_specs=pl.BlockSpec((bt, F), lambda i: (i, 0)),
        out_shape=jax.ShapeDtypeStruct((T, F), _f32),
        compiler_params=pltpu.CompilerParams(
            dimension_semantics=("parallel",)),
    )(sbf, gathered, w_sbf1, w_sbf2)


_sc_mesh = plsc.VectorSubcoreMesh(
    core_axis_name="c", subcore_axis_name="s",
    num_cores=NCORE, num_subcores=NSUB)


@functools.partial(
    pl.kernel,
    out_type=jax.ShapeDtypeStruct((T, F), _f32),
    mesh=_sc_mesh,
    scratch_types=[
        pltpu.VMEM((GB,), jnp.int32),
        pltpu.VMEM((GB, F), _f32),
        pltpu.SemaphoreType.DMA,
    ],
    compiler_params=pltpu.CompilerParams(use_tc_tiling_on_sc=False, needs_layout_passes=False),
)
def _sc_gather(xkj_hbm, idxkj_hbm, out_hbm, idx_v, rows_v, sem):
    wid = lax.axis_index("s") * NCORE + lax.axis_index("c")
    base = wid * TW

    def blk(g, carry):
        off = base + g * GB
        pltpu.sync_copy(idxkj_hbm.at[pl.ds(off, GB)], idx_v)
        pltpu.async_copy(xkj_hbm.at[idx_v], rows_v, sem).wait()
        pltpu.sync_copy(rows_v, out_hbm.at[pl.ds(off, GB)])
        return carry

    lax.fori_loop(0, TW // GB, blk, jnp.int32(0))


@functools.partial(
    pl.kernel,
    out_type=jax.ShapeDtypeStruct((EP, F), _f32),
    mesh=_sc_mesh,
    scratch_types=[
        pltpu.VMEM((MB,), jnp.int32),
        pltpu.VMEM((NGRP, 128), jnp.int32),
        pltpu.VMEM((NGRP, 128), jnp.int32),
        pltpu.VMEM((128, F), _f32),
        pltpu.VMEM((16, F), _f32),
        pltpu.VMEM((16,), jnp.int32),
        pltpu.VMEM((16,), jnp.int32),
        pltpu.VMEM((C,), jnp.int32),
        pltpu.VMEM_SHARED((C + 128, F), _f32),
        pltpu.SemaphoreType.DMA,
    ],
    compiler_params=pltpu.CompilerParams(use_tc_tiling_on_sc=False, needs_layout_passes=False),
)
def _sc_scatter(prod_hbm, idxji_hbm, zeros_hbm, out_hbm,
                stage, tix2d, dst2d, prows, b16, tixb, dstb, cntarr, acc, sem):
    cid = lax.axis_index("c")
    sid = lax.axis_index("s")
    iot = jnp.arange(16, dtype=jnp.int32)
    zero16 = jnp.zeros((16,), jnp.int32)

    def zcnt(i, carry):
        cntarr[pl.ds(i * 16, 16)] = zero16
        return carry
    lax.fori_loop(0, C // 16, zcnt, jnp.int32(0))

    occ0, _ = plsc.scan_count(zero16)
    occ_base = jnp.sum(jnp.where(iot == 0, occ0, 0))
    occ_off = 1 - occ_base

    def fire_b(tixv, dstv, valid):
        occ2, _ = plsc.scan_count(jnp.where(valid, dstv, -1), mask=valid)
        occ2 = occ2 + occ_off
        mx = jnp.max(jnp.where(valid, occ2, 0))

        def rnd(r, carry):
            mr = valid & (occ2 == r + 1)
            tixb[...] = jnp.where(mr, tixv, iot)
            dstb[...] = jnp.where(mr, dstv, C + iot)
            pltpu.async_copy(prod_hbm.at[tixb], b16, sem).wait()
            pltpu.sync_copy(b16, acc.at[dstb], add=True)
            return carry

        lax.fori_loop(0, mx, rnd, jnp.int32(0))

    def process_b(nb):
        def bvreg(k, carry):
            row = AROWS + lax.shift_right_logical(k, 3)
            cb = lax.bitwise_and(k, 7) * 16
            dstv = dst2d[row, pl.ds(cb, 16)]
            tixv = tix2d[row, pl.ds(cb, 16)]
            valid = (k * 16 + iot) < nb
            fire_b(tixv, dstv, valid)
            return carry

        nv = lax.shift_right_logical(nb + 15, 4)
        lax.fori_loop(0, nv, bvreg, jnp.int32(0))

    def do_pass(g, carry):
        lo = (KCH * cid + g) * C
        pltpu.sync_copy(zeros_hbm.at[pl.ds(0, ZROWS)],
                        acc.at[pl.ds(sid * ZROWS, ZROWS)])
        plsc.subcore_barrier()

        def macro(mb, carry_mb):
            base = sid * TS + mb * MB
            pltpu.sync_copy(idxji_hbm.at[pl.ds(base, MB)], stage)

            def scan16(i, cnt):
                na, nb = cnt
                v = stage[pl.ds(i * 16, 16)]
                m = (v >= lo) & (v < lo + C)
                dst = jnp.where(m, v - lo, 0)
                occ, lastm = plsc.scan_count(jnp.where(m, dst, -1), mask=m)
                prev = plsc.load_gather(cntarr, [dst])
                tocc = prev + occ + occ_off
                plsc.store_scatter(cntarr, [dst], tocc, mask=lastm & m)
                tv = base + i * 16 + iot
                ma = m & (tocc == 1)
                slot = na + plsc.cumsum(jnp.where(ma, 1, 0)) - 1
                row = lax.shift_right_logical(slot, 7)
                col = lax.bitwise_and(slot, 127)
                plsc.store_scatter(dst2d, [row, col], dst, mask=ma)
                plsc.store_scatter(tix2d, [row, col], tv, mask=ma)
                mb2 = m & (tocc > 1)
                slotb = nb + plsc.cumsum(jnp.where(mb2, 1, 0)) - 1
                rowb = AROWS + lax.shift_right_logical(slotb, 7)
                colb = lax.bitwise_and(slotb, 127)
                plsc.store_scatter(dst2d, [rowb, colb], dst, mask=mb2)
                plsc.store_scatter(tix2d, [rowb, colb], tv, mask=mb2)
                na = na + jnp.sum(jnp.where(ma, 1, 0))
                nb = nb + jnp.sum(jnp.where(mb2, 1, 0))

                def flush(nbf):
                    process_b(nbf)
                    return jnp.int32(0)

                nb = lax.cond(nb > (NGRP - AROWS) * 128 - 16, flush,
                              lambda x: x, nb)
                return (na, nb)

            na, nb = lax.fori_loop(0, MB // 16, scan16,
                                   (jnp.int32(0), jnp.int32(0)))

            r_last = jnp.full((16,), lax.shift_right_logical(na, 7), jnp.int32)
            bcol = lax.bitwise_and(na, 127)
            for j in range(8):
                colv = iot + j * 16
                mt = colv >= bcol
                plsc.store_scatter(dst2d, [r_last, colv], C + colv, mask=mt)
                plsc.store_scatter(tix2d, [r_last, colv], colv, mask=mt)

            n_grp = lax.shift_right_logical(na + 127, 7)

            def grp(g2, carry2):
                pltpu.async_copy(prod_hbm.at[tix2d.at[g2]], prows, sem).wait()
                pltpu.sync_copy(prows, acc.at[dst2d.at[g2]], add=True)
                return carry2

            lax.fori_loop(0, n_grp, grp, jnp.int32(0))
            process_b(nb)

            def zrow(k, carry):
                row = lax.shift_right_logical(k, 3)
                cb = lax.bitwise_and(k, 7) * 16
                dstv = dst2d[row, pl.ds(cb, 16)]
                valid = (k * 16 + iot) < na
                plsc.store_scatter(cntarr, [jnp.where(valid, dstv, 0)],
                                   zero16, mask=valid)
                return carry

            lax.fori_loop(0, lax.shift_right_logical(na + 15, 4), zrow,
                          jnp.int32(0))
            return carry_mb

        lax.fori_loop(0, NMB, macro, jnp.int32(0))
        plsc.subcore_barrier()
        pltpu.sync_copy(acc.at[pl.ds(sid * DRAIN, DRAIN)],
                        out_hbm.at[pl.ds(lo + sid * DRAIN, DRAIN)])
        plsc.subcore_barrier()
        return carry

    lax.fori_loop(0, KCH, do_pass, jnp.int32(0))


def _final_body(agg_ref, xji_ref, wup_ref, out_ref):
    u = jnp.dot(agg_ref[...], wup_ref[...], preferred_element_type=_f32)
    out_ref[...] = xji_ref[...] + _swish(u)


def _final(agg, x_ji, w_up):
    be = 2000
    grid = (E // be,)
    return pl.pallas_call(
        _final_body,
        grid=grid,
        in_specs=[
            pl.BlockSpec((be, F), lambda i: (i, 0)),
            pl.BlockSpec((be, H), lambda i: (i, 0)),
            pl.BlockSpec((F, H), lambda i: (0, 0)),
        ],
        out_specs=pl.BlockSpec((be, H), lambda i: (i, 0)),
        out_shape=jax.ShapeDtypeStruct((E, H), _f32),
        compiler_params=pltpu.CompilerParams(
            dimension_semantics=("parallel",)),
    )(agg, x_ji, w_up)


def kernel(x1, x2, rbf0, sbf, t, idx_kj, idx_ji,
           W_rbf1, W_rbf2, W_sbf1, W_sbf2, W_kj, b_kj, W_ji, b_ji,
           W_down, W_up):
    idx_kj = idx_kj.astype(jnp.int32)
    idx_ji = idx_ji.astype(jnp.int32)
    x_ji, xkj = _edges(x1, rbf0, W_ji, b_ji.reshape(1, H), W_kj,
                       b_kj.reshape(1, H), W_rbf1, W_rbf2, W_down)
    gathered = _sc_gather(xkj, idx_kj)
    prod = _prod(sbf, gathered, W_sbf1, W_sbf2)
    zeros = jnp.zeros((ZROWS, F), _f32)
    agg = _sc_scatter(prod, idx_ji, zeros)
    return _final(agg, x_ji, W_up)


# speedup vs baseline: 1.2474x; 1.2474x over previous
import functools

import jax
import jax.numpy as jnp
from jax import lax
from jax.experimental import pallas as pl
from jax.experimental.pallas import tpu as pltpu
from jax.experimental.pallas import tpu_sc as plsc

E = 160000
T = 480000
H = 128
F = 64
NR = 6
NSR = 42

NCORE = 2
NSUB = 16

C = 20000
KCH = 4
EP = NCORE * KCH * C
TS = T // NSUB
TW = T // (NCORE * NSUB)
GB = 600
MB = 3000
NMB = TS // MB
AROWS = 24
NGRP = 32
ZROWS = (C + 128) // NSUB
DRAIN = C // NSUB

_f32 = jnp.float32


def _swish(x):
    return x * jax.nn.sigmoid(x)


def _edges_body(x1_ref, rbf0_ref, wji_ref, bji_ref, wkj_ref, bkj_ref,
                wr1_ref, wr2_ref, wd_ref, xji_ref, xkj_ref):
    x1 = x1_ref[...]
    a = jnp.dot(x1, wji_ref[...], preferred_element_type=_f32) + bji_ref[...]
    xji_ref[...] = _swish(a)
    b = jnp.dot(x1, wkj_ref[...], preferred_element_type=_f32) + bkj_ref[...]
    r1 = jnp.dot(rbf0_ref[...], wr1_ref[...], preferred_element_type=_f32)
    rbf = jnp.dot(r1, wr2_ref[...], preferred_element_type=_f32)
    c = _swish(b) * rbf
    d = jnp.dot(c, wd_ref[...], preferred_element_type=_f32)
    xkj_ref[...] = _swish(d)


def _edges(x1, rbf0, w_ji, b_ji, w_kj, b_kj, w_rbf1, w_rbf2, w_down):
    be = 2000
    grid = (E // be,)
    full = lambda shape: pl.BlockSpec(shape, lambda i: (0, 0))
    return pl.pallas_call(
        _edges_body,
        grid=grid,
        in_specs=[
            pl.BlockSpec((be, H), lambda i: (i, 0)),
            pl.BlockSpec((be, NR), lambda i: (i, 0)),
            full((H, H)), full((1, H)), full((H, H)), full((1, H)),
            full((NR, 8)), full((8, H)), full((H, F)),
        ],
        out_specs=[
            pl.BlockSpec((be, H), lambda i: (i, 0)),
            pl.BlockSpec((be, F), lambda i: (i, 0)),
        ],
        out_shape=[
            jax.ShapeDtypeStruct((E, H), _f32),
            jax.ShapeDtypeStruct((E, F), _f32),
        ],
        compiler_params=pltpu.CompilerParams(
            dimension_semantics=("parallel",)),
    )(x1, rbf0, w_ji, b_ji, w_kj, b_kj, w_rbf1, w_rbf2, w_down)


def _prod_body(sbf_ref, g_ref, w1_ref, w2_ref, out_ref):
    s1 = jnp.dot(sbf_ref[...], w1_ref[...], preferred_element_type=_f32)
    s = jnp.dot(s1, w2_ref[...], preferred_element_type=_f32)
    out_ref[...] = s * g_ref[...]


def _prod(sbf2, g2, w1b, w2b):
    bt2 = 2000
    grid = (T // 2 // bt2,)
    full = lambda shape: pl.BlockSpec(shape, lambda i: (0, 0))
    return pl.pallas_call(
        _prod_body,
        grid=grid,
        in_specs=[
            pl.BlockSpec((bt2, 2 * NSR), lambda i: (i, 0)),
            pl.BlockSpec((bt2, 2 * F), lambda i: (i, 0)),
            full((2 * NSR, 2 * F)), full((2 * F, 2 * F)),
        ],
        out_specs=pl.BlockSpec((bt2, 2 * F), lambda i: (i, 0)),
        out_shape=jax.ShapeDtypeStruct((T // 2, 2 * F), _f32),
        compiler_params=pltpu.CompilerParams(
            dimension_semantics=("parallel",)),
    )(sbf2, g2, w1b, w2b)


_sc_mesh = plsc.VectorSubcoreMesh(
    core_axis_name="c", subcore_axis_name="s",
    num_cores=NCORE, num_subcores=NSUB)


@functools.partial(
    pl.kernel,
    out_type=jax.ShapeDtypeStruct((T, F), _f32),
    mesh=_sc_mesh,
    scratch_types=[
        pltpu.VMEM((GB,), jnp.int32),
        pltpu.VMEM((GB, F), _f32),
        pltpu.SemaphoreType.DMA,
    ],
    compiler_params=pltpu.CompilerParams(use_tc_tiling_on_sc=False, needs_layout_passes=False),
)
def _sc_gather(xkj_hbm, idxkj_hbm, out_hbm, idx_v, rows_v, sem):
    wid = lax.axis_index("s") * NCORE + lax.axis_index("c")
    base = wid * TW

    def blk(g, carry):
        off = base + g * GB
        pltpu.sync_copy(idxkj_hbm.at[pl.ds(off, GB)], idx_v)
        pltpu.async_copy(xkj_hbm.at[idx_v], rows_v, sem).wait()
        pltpu.sync_copy(rows_v, out_hbm.at[pl.ds(off, GB)])
        return carry

    lax.fori_loop(0, TW // GB, blk, jnp.int32(0))


@functools.partial(
    pl.kernel,
    out_type=jax.ShapeDtypeStruct((EP, F), _f32),
    mesh=_sc_mesh,
    scratch_types=[
        pltpu.VMEM((MB,), jnp.int32),
        pltpu.VMEM((NGRP, 128), jnp.int32),
        pltpu.VMEM((NGRP, 128), jnp.int32),
        pltpu.VMEM((128, F), _f32),
        pltpu.VMEM((16, F), _f32),
        pltpu.VMEM((16,), jnp.int32),
        pltpu.VMEM((16,), jnp.int32),
        pltpu.VMEM((C,), jnp.int32),
        pltpu.VMEM_SHARED((C + 128, F), _f32),
        pltpu.SemaphoreType.DMA,
    ],
    compiler_params=pltpu.CompilerParams(use_tc_tiling_on_sc=False, needs_layout_passes=False),
)
def _sc_scatter(prod_hbm, idxji_hbm, zeros_hbm, out_hbm,
                stage, tix2d, dst2d, prows, b16, tixb, dstb, cntarr, acc, sem):
    cid = lax.axis_index("c")
    sid = lax.axis_index("s")
    iot = jnp.arange(16, dtype=jnp.int32)
    zero16 = jnp.zeros((16,), jnp.int32)

    def zcnt(i, carry):
        cntarr[pl.ds(i * 16, 16)] = zero16
        return carry
    lax.fori_loop(0, C // 16, zcnt, jnp.int32(0))

    occ0, _ = plsc.scan_count(zero16)
    occ_base = jnp.sum(jnp.where(iot == 0, occ0, 0))
    occ_off = 1 - occ_base

    def fire_b(tixv, dstv, valid):
        occ2, _ = plsc.scan_count(jnp.where(valid, dstv, -1), mask=valid)
        occ2 = occ2 + occ_off
        mx = jnp.max(jnp.where(valid, occ2, 0))

        def rnd(r, carry):
            mr = valid & (occ2 == r + 1)
            tixb[...] = jnp.where(mr, tixv, iot)
            dstb[...] = jnp.where(mr, dstv, C + iot)
            pltpu.async_copy(prod_hbm.at[tixb], b16, sem).wait()
            pltpu.sync_copy(b16, acc.at[dstb], add=True)
            return carry

        lax.fori_loop(0, mx, rnd, jnp.int32(0))

    def process_b(nb):
        def bvreg(k, carry):
            row = AROWS + lax.shift_right_logical(k, 3)
            cb = lax.bitwise_and(k, 7) * 16
            dstv = dst2d[row, pl.ds(cb, 16)]
            tixv = tix2d[row, pl.ds(cb, 16)]
            valid = (k * 16 + iot) < nb
            fire_b(tixv, dstv, valid)
            return carry

        nv = lax.shift_right_logical(nb + 15, 4)
        lax.fori_loop(0, nv, bvreg, jnp.int32(0))

    def do_pass(g, carry):
        lo = (KCH * cid + g) * C
        pltpu.sync_copy(zeros_hbm.at[pl.ds(0, ZROWS)],
                        acc.at[pl.ds(sid * ZROWS, ZROWS)])
        plsc.subcore_barrier()

        def macro(mb, carry_mb):
            base = sid * TS + mb * MB
            pltpu.sync_copy(idxji_hbm.at[pl.ds(base, MB)], stage)

            def scan16(i, cnt):
                na, nb = cnt
                v = stage[pl.ds(i * 16, 16)]
                m = (v >= lo) & (v < lo + C)
                dst = jnp.where(m, v - lo, 0)
                occ, lastm = plsc.scan_count(jnp.where(m, dst, -1), mask=m)
                prev = plsc.load_gather(cntarr, [dst])
                tocc = prev + occ + occ_off
                plsc.store_scatter(cntarr, [dst], tocc, mask=lastm & m)
                tv = base + i * 16 + iot
                ma = m & (tocc == 1)
                slot = na + plsc.cumsum(jnp.where(ma, 1, 0)) - 1
                row = lax.shift_right_logical(slot, 7)
                col = lax.bitwise_and(slot, 127)
                plsc.store_scatter(dst2d, [row, col], dst, mask=ma)
                plsc.store_scatter(tix2d, [row, col], tv, mask=ma)
                mb2 = m & (tocc > 1)
                slotb = nb + plsc.cumsum(jnp.where(mb2, 1, 0)) - 1
                rowb = AROWS + lax.shift_right_logical(slotb, 7)
                colb = lax.bitwise_and(slotb, 127)
                plsc.store_scatter(dst2d, [rowb, colb], dst, mask=mb2)
                plsc.store_scatter(tix2d, [rowb, colb], tv, mask=mb2)
                na = na + jnp.sum(jnp.where(ma, 1, 0))
                nb = nb + jnp.sum(jnp.where(mb2, 1, 0))

                def flush(nbf):
                    process_b(nbf)
                    return jnp.int32(0)

                nb = lax.cond(nb > (NGRP - AROWS) * 128 - 16, flush,
                              lambda x: x, nb)
                return (na, nb)

            na, nb = lax.fori_loop(0, MB // 16, scan16,
                                   (jnp.int32(0), jnp.int32(0)))

            r_last = jnp.full((16,), lax.shift_right_logical(na, 7), jnp.int32)
            bcol = lax.bitwise_and(na, 127)
            for j in range(8):
                colv = iot + j * 16
                mt = colv >= bcol
                plsc.store_scatter(dst2d, [r_last, colv], C + colv, mask=mt)
                plsc.store_scatter(tix2d, [r_last, colv], colv, mask=mt)

            n_grp = lax.shift_right_logical(na + 127, 7)

            def grp(g2, carry2):
                pltpu.async_copy(prod_hbm.at[tix2d.at[g2]], prows, sem).wait()
                pltpu.sync_copy(prows, acc.at[dst2d.at[g2]], add=True)
                return carry2

            lax.fori_loop(0, n_grp, grp, jnp.int32(0))
            process_b(nb)

            def zrow(k, carry):
                row = lax.shift_right_logical(k, 3)
                cb = lax.bitwise_and(k, 7) * 16
                dstv = dst2d[row, pl.ds(cb, 16)]
                valid = (k * 16 + iot) < na
                plsc.store_scatter(cntarr, [jnp.where(valid, dstv, 0)],
                                   zero16, mask=valid)
                return carry

            lax.fori_loop(0, lax.shift_right_logical(na + 15, 4), zrow,
                          jnp.int32(0))
            return carry_mb

        lax.fori_loop(0, NMB, macro, jnp.int32(0))
        plsc.subcore_barrier()
        pltpu.sync_copy(acc.at[pl.ds(sid * DRAIN, DRAIN)],
                        out_hbm.at[pl.ds(lo + sid * DRAIN, DRAIN)])
        plsc.subcore_barrier()
        return carry

    lax.fori_loop(0, KCH, do_pass, jnp.int32(0))


def _final_body(agg_ref, xji_ref, wup_ref, out_ref):
    u = jnp.dot(agg_ref[...], wup_ref[...], preferred_element_type=_f32)
    out_ref[...] = xji_ref[...] + _swish(u)


def _final(agg, x_ji, w_up):
    be = 2000
    grid = (E // be,)
    return pl.pallas_call(
        _final_body,
        grid=grid,
        in_specs=[
            pl.BlockSpec((be, F), lambda i: (i, 0)),
            pl.BlockSpec((be, H), lambda i: (i, 0)),
            pl.BlockSpec((F, H), lambda i: (0, 0)),
        ],
        out_specs=pl.BlockSpec((be, H), lambda i: (i, 0)),
        out_shape=jax.ShapeDtypeStruct((E, H), _f32),
        compiler_params=pltpu.CompilerParams(
            dimension_semantics=("parallel",)),
    )(agg, x_ji, w_up)


def kernel(x1, x2, rbf0, sbf, t, idx_kj, idx_ji,
           W_rbf1, W_rbf2, W_sbf1, W_sbf2, W_kj, b_kj, W_ji, b_ji,
           W_down, W_up):
    idx_kj = idx_kj.astype(jnp.int32)
    idx_ji = idx_ji.astype(jnp.int32)
    x_ji, xkj = _edges(x1, rbf0, W_ji, b_ji.reshape(1, H), W_kj,
                       b_kj.reshape(1, H), W_rbf1, W_rbf2, W_down)
    gathered = _sc_gather(xkj, idx_kj)
    w1b = jnp.block([[W_sbf1, jnp.zeros((NSR, F), _f32)],
                     [jnp.zeros((NSR, F), _f32), W_sbf1]])
    w2b = jnp.block([[W_sbf2, jnp.zeros((F, F), _f32)],
                     [jnp.zeros((F, F), _f32), W_sbf2]])
    sbf2 = sbf.reshape(T // 2, 2 * NSR)
    prod2 = _prod(sbf2, gathered.reshape(T // 2, 2 * F), w1b, w2b)
    zeros = jnp.zeros((ZROWS, F), _f32)
    agg = _sc_scatter(prod2.reshape(T, F), idx_ji, zeros)
    return _final(agg, x_ji, W_up)


# speedup vs baseline: 1.2663x; 1.0152x over previous
import functools

import jax
import jax.numpy as jnp
from jax import lax
from jax.experimental import pallas as pl
from jax.experimental.pallas import tpu as pltpu
from jax.experimental.pallas import tpu_sc as plsc

E = 160000
T = 480000
H = 128
F = 64
NR = 6
NSR = 42

NCORE = 2
NSUB = 16

C = 20000
KCH = 4
EP = NCORE * KCH * C
TS = T // NSUB
TW = T // (NCORE * NSUB)
GB = 1000
MB = 5000
NMB = TS // MB
AROWS = 40
NGRP = 48
ZROWS = (C + 128) // NSUB
DRAIN = C // NSUB

_f32 = jnp.float32


def _swish(x):
    return x * jax.nn.sigmoid(x)


def _edges_body(x1_ref, rbf0_ref, wji_ref, bji_ref, wkj_ref, bkj_ref,
                wr1_ref, wr2_ref, wd_ref, xji_ref, xkj_ref):
    x1 = x1_ref[...]
    a = jnp.dot(x1, wji_ref[...], preferred_element_type=_f32) + bji_ref[...]
    xji_ref[...] = _swish(a)
    b = jnp.dot(x1, wkj_ref[...], preferred_element_type=_f32) + bkj_ref[...]
    r1 = jnp.dot(rbf0_ref[...], wr1_ref[...], preferred_element_type=_f32)
    rbf = jnp.dot(r1, wr2_ref[...], preferred_element_type=_f32)
    c = _swish(b) * rbf
    d = jnp.dot(c, wd_ref[...], preferred_element_type=_f32)
    xkj_ref[...] = _swish(d)


def _edges(x1, rbf0, w_ji, b_ji, w_kj, b_kj, w_rbf1, w_rbf2, w_down):
    be = 2000
    grid = (E // be,)
    full = lambda shape: pl.BlockSpec(shape, lambda i: (0, 0))
    return pl.pallas_call(
        _edges_body,
        grid=grid,
        in_specs=[
            pl.BlockSpec((be, H), lambda i: (i, 0)),
            pl.BlockSpec((be, NR), lambda i: (i, 0)),
            full((H, H)), full((1, H)), full((H, H)), full((1, H)),
            full((NR, 8)), full((8, H)), full((H, F)),
        ],
        out_specs=[
            pl.BlockSpec((be, H), lambda i: (i, 0)),
            pl.BlockSpec((be, F), lambda i: (i, 0)),
        ],
        out_shape=[
            jax.ShapeDtypeStruct((E, H), _f32),
            jax.ShapeDtypeStruct((E, F), _f32),
        ],
        compiler_params=pltpu.CompilerParams(
            dimension_semantics=("parallel",)),
    )(x1, rbf0, w_ji, b_ji, w_kj, b_kj, w_rbf1, w_rbf2, w_down)


def _prod_body(sbf_ref, g_ref, w1_ref, w2_ref, out_ref):
    s1 = jnp.dot(sbf_ref[...], w1_ref[...], preferred_element_type=_f32)
    s = jnp.dot(s1, w2_ref[...], preferred_element_type=_f32)
    out_ref[...] = s * g_ref[...]


def _prod(sbf2, g2, w1b, w2b):
    bt2 = 2000
    grid = (T // 2 // bt2,)
    full = lambda shape: pl.BlockSpec(shape, lambda i: (0, 0))
    return pl.pallas_call(
        _prod_body,
        grid=grid,
        in_specs=[
            pl.BlockSpec((bt2, 2 * NSR), lambda i: (i, 0)),
            pl.BlockSpec((bt2, 2 * F), lambda i: (i, 0)),
            full((2 * NSR, 2 * F)), full((2 * F, 2 * F)),
        ],
        out_specs=pl.BlockSpec((bt2, 2 * F), lambda i: (i, 0)),
        out_shape=jax.ShapeDtypeStruct((T // 2, 2 * F), _f32),
        compiler_params=pltpu.CompilerParams(
            dimension_semantics=("parallel",)),
    )(sbf2, g2, w1b, w2b)


_sc_mesh = plsc.VectorSubcoreMesh(
    core_axis_name="c", subcore_axis_name="s",
    num_cores=NCORE, num_subcores=NSUB)


@functools.partial(
    pl.kernel,
    out_type=jax.ShapeDtypeStruct((T, F), _f32),
    mesh=_sc_mesh,
    scratch_types=[
        pltpu.VMEM((GB,), jnp.int32),
        pltpu.VMEM((GB, F), _f32),
        pltpu.SemaphoreType.DMA,
    ],
    compiler_params=pltpu.CompilerParams(use_tc_tiling_on_sc=False, needs_layout_passes=False),
)
def _sc_gather(xkj_hbm, idxkj_hbm, out_hbm, idx_v, rows_v, sem):
    wid = lax.axis_index("s") * NCORE + lax.axis_index("c")
    base = wid * TW

    def blk(g, carry):
        off = base + g * GB
        pltpu.sync_copy(idxkj_hbm.at[pl.ds(off, GB)], idx_v)
        pltpu.async_copy(xkj_hbm.at[idx_v], rows_v, sem).wait()
        pltpu.sync_copy(rows_v, out_hbm.at[pl.ds(off, GB)])
        return carry

    lax.fori_loop(0, TW // GB, blk, jnp.int32(0))


@functools.partial(
    pl.kernel,
    out_type=jax.ShapeDtypeStruct((EP, F), _f32),
    mesh=_sc_mesh,
    scratch_types=[
        pltpu.VMEM((MB,), jnp.int32),
        pltpu.VMEM((NGRP, 128), jnp.int32),
        pltpu.VMEM((NGRP, 128), jnp.int32),
        pltpu.VMEM((128, F), _f32),
        pltpu.VMEM((16, F), _f32),
        pltpu.VMEM((16,), jnp.int32),
        pltpu.VMEM((16,), jnp.int32),
        pltpu.VMEM((C,), jnp.int32),
        pltpu.VMEM_SHARED((C + 128, F), _f32),
        pltpu.SemaphoreType.DMA,
    ],
    compiler_params=pltpu.CompilerParams(use_tc_tiling_on_sc=False, needs_layout_passes=False),
)
def _sc_scatter(prod_hbm, idxji_hbm, zeros_hbm, out_hbm,
                stage, tix2d, dst2d, prows, b16, tixb, dstb, cntarr, acc, sem):
    cid = lax.axis_index("c")
    sid = lax.axis_index("s")
    iot = jnp.arange(16, dtype=jnp.int32)
    zero16 = jnp.zeros((16,), jnp.int32)

    def zcnt(i, carry):
        cntarr[pl.ds(i * 16, 16)] = zero16
        return carry
    lax.fori_loop(0, C // 16, zcnt, jnp.int32(0))

    occ0, _ = plsc.scan_count(zero16)
    occ_base = jnp.sum(jnp.where(iot == 0, occ0, 0))
    occ_off = 1 - occ_base

    def fire_b(tixv, dstv, valid):
        occ2, _ = plsc.scan_count(jnp.where(valid, dstv, -1), mask=valid)
        occ2 = occ2 + occ_off
        mx = jnp.max(jnp.where(valid, occ2, 0))

        def rnd(r, carry):
            mr = valid & (occ2 == r + 1)
            tixb[...] = jnp.where(mr, tixv, iot)
            dstb[...] = jnp.where(mr, dstv, C + iot)
            pltpu.async_copy(prod_hbm.at[tixb], b16, sem).wait()
            pltpu.sync_copy(b16, acc.at[dstb], add=True)
            return carry

        lax.fori_loop(0, mx, rnd, jnp.int32(0))

    def process_b(nb):
        def bvreg(k, carry):
            row = AROWS + lax.shift_right_logical(k, 3)
            cb = lax.bitwise_and(k, 7) * 16
            dstv = dst2d[row, pl.ds(cb, 16)]
            tixv = tix2d[row, pl.ds(cb, 16)]
            valid = (k * 16 + iot) < nb
            fire_b(tixv, dstv, valid)
            return carry

        nv = lax.shift_right_logical(nb + 15, 4)
        lax.fori_loop(0, nv, bvreg, jnp.int32(0))

    def do_pass(g, carry):
        lo = (KCH * cid + g) * C
        pltpu.sync_copy(zeros_hbm.at[pl.ds(0, ZROWS)],
                        acc.at[pl.ds(sid * ZROWS, ZROWS)])
        plsc.subcore_barrier()

        def macro(mb, carry_mb):
            base = sid * TS + mb * MB
            pltpu.sync_copy(idxji_hbm.at[pl.ds(base, MB)], stage)

            def scan16(i, cnt):
                na, nb = cnt
                v = stage[pl.ds(i * 16, 16)]
                m = (v >= lo) & (v < lo + C)
                dst = jnp.where(m, v - lo, 0)
                occ, lastm = plsc.scan_count(jnp.where(m, dst, -1), mask=m)
                prev = plsc.load_gather(cntarr, [dst])
                tocc = prev + occ + occ_off
                plsc.store_scatter(cntarr, [dst], tocc, mask=lastm & m)
                tv = base + i * 16 + iot
                ma = m & (tocc == 1)
                slot = na + plsc.cumsum(jnp.where(ma, 1, 0)) - 1
                row = lax.shift_right_logical(slot, 7)
                col = lax.bitwise_and(slot, 127)
                plsc.store_scatter(dst2d, [row, col], dst, mask=ma)
                plsc.store_scatter(tix2d, [row, col], tv, mask=ma)
                mb2 = m & (tocc > 1)
                slotb = nb + plsc.cumsum(jnp.where(mb2, 1, 0)) - 1
                rowb = AROWS + lax.shift_right_logical(slotb, 7)
                colb = lax.bitwise_and(slotb, 127)
                plsc.store_scatter(dst2d, [rowb, colb], dst, mask=mb2)
                plsc.store_scatter(tix2d, [rowb, colb], tv, mask=mb2)
                na = na + jnp.sum(jnp.where(ma, 1, 0))
                nb = nb + jnp.sum(jnp.where(mb2, 1, 0))

                def flush(nbf):
                    process_b(nbf)
                    return jnp.int32(0)

                nb = lax.cond(nb > (NGRP - AROWS) * 128 - 16, flush,
                              lambda x: x, nb)
                return (na, nb)

            na, nb = lax.fori_loop(0, MB // 16, scan16,
                                   (jnp.int32(0), jnp.int32(0)))

            r_last = jnp.full((16,), lax.shift_right_logical(na, 7), jnp.int32)
            bcol = lax.bitwise_and(na, 127)
            for j in range(8):
                colv = iot + j * 16
                mt = colv >= bcol
                plsc.store_scatter(dst2d, [r_last, colv], C + colv, mask=mt)
                plsc.store_scatter(tix2d, [r_last, colv], colv, mask=mt)

            n_grp = lax.shift_right_logical(na + 127, 7)

            def grp(g2, carry2):
                pltpu.async_copy(prod_hbm.at[tix2d.at[g2]], prows, sem).wait()
                pltpu.sync_copy(prows, acc.at[dst2d.at[g2]], add=True)
                return carry2

            lax.fori_loop(0, n_grp, grp, jnp.int32(0))
            process_b(nb)

            def zrow(k, carry):
                row = lax.shift_right_logical(k, 3)
                cb = lax.bitwise_and(k, 7) * 16
                dstv = dst2d[row, pl.ds(cb, 16)]
                valid = (k * 16 + iot) < na
                plsc.store_scatter(cntarr, [jnp.where(valid, dstv, 0)],
                                   zero16, mask=valid)
                return carry

            lax.fori_loop(0, lax.shift_right_logical(na + 15, 4), zrow,
                          jnp.int32(0))
            return carry_mb

        lax.fori_loop(0, NMB, macro, jnp.int32(0))
        plsc.subcore_barrier()
        pltpu.sync_copy(acc.at[pl.ds(sid * DRAIN, DRAIN)],
                        out_hbm.at[pl.ds(lo + sid * DRAIN, DRAIN)])
        plsc.subcore_barrier()
        return carry

    lax.fori_loop(0, KCH, do_pass, jnp.int32(0))


def _final_body(agg_ref, xji_ref, wup_ref, out_ref):
    u = jnp.dot(agg_ref[...], wup_ref[...], preferred_element_type=_f32)
    out_ref[...] = xji_ref[...] + _swish(u)


def _final(agg, x_ji, w_up):
    be = 2000
    grid = (E // be,)
    return pl.pallas_call(
        _final_body,
        grid=grid,
        in_specs=[
            pl.BlockSpec((be, F), lambda i: (i, 0)),
            pl.BlockSpec((be, H), lambda i: (i, 0)),
            pl.BlockSpec((F, H), lambda i: (0, 0)),
        ],
        out_specs=pl.BlockSpec((be, H), lambda i: (i, 0)),
        out_shape=jax.ShapeDtypeStruct((E, H), _f32),
        compiler_params=pltpu.CompilerParams(
            dimension_semantics=("parallel",)),
    )(agg, x_ji, w_up)


def kernel(x1, x2, rbf0, sbf, t, idx_kj, idx_ji,
           W_rbf1, W_rbf2, W_sbf1, W_sbf2, W_kj, b_kj, W_ji, b_ji,
           W_down, W_up):
    idx_kj = idx_kj.astype(jnp.int32)
    idx_ji = idx_ji.astype(jnp.int32)
    x_ji, xkj = _edges(x1, rbf0, W_ji, b_ji.reshape(1, H), W_kj,
                       b_kj.reshape(1, H), W_rbf1, W_rbf2, W_down)
    gathered = _sc_gather(xkj, idx_kj)
    w1b = jnp.block([[W_sbf1, jnp.zeros((NSR, F), _f32)],
                     [jnp.zeros((NSR, F), _f32), W_sbf1]])
    w2b = jnp.block([[W_sbf2, jnp.zeros((F, F), _f32)],
                     [jnp.zeros((F, F), _f32), W_sbf2]])
    sbf2 = sbf.reshape(T // 2, 2 * NSR)
    prod2 = _prod(sbf2, gathered.reshape(T // 2, 2 * F), w1b, w2b)
    zeros = jnp.zeros((ZROWS, F), _f32)
    agg = _sc_scatter(prod2.reshape(T, F), idx_ji, zeros)
    return _final(agg, x_ji, W_up)


# speedup vs baseline: 1.3217x; 1.0437x over previous
import functools

import jax
import jax.numpy as jnp
from jax import lax
from jax.experimental import pallas as pl
from jax.experimental.pallas import tpu as pltpu
from jax.experimental.pallas import tpu_sc as plsc

E = 160000
T = 480000
H = 128
F = 64
NR = 6
NSR = 42

NCORE = 2
NSUB = 16

C = 20000
KCH = 4
EP = NCORE * KCH * C
TS = T // NSUB
TW = T // (NCORE * NSUB)
GB = 1000
MB = 5000
NMB = TS // MB
AROWS = 40
NGRP = 48
ZROWS = (C + 128) // NSUB
DRAIN = C // NSUB

_f32 = jnp.float32


def _swish(x):
    return x * jax.nn.sigmoid(x)


def _edges_body(x1_ref, rbf0_ref, wji_ref, bji_ref, wkj_ref, bkj_ref,
                wr1_ref, wr2_ref, wd_ref, xji_ref, xkj_ref):
    x1 = x1_ref[...]
    a = jnp.dot(x1, wji_ref[...], preferred_element_type=_f32) + bji_ref[...]
    xji_ref[...] = _swish(a)
    b = jnp.dot(x1, wkj_ref[...], preferred_element_type=_f32) + bkj_ref[...]
    r1 = jnp.dot(rbf0_ref[...], wr1_ref[...], preferred_element_type=_f32)
    rbf = jnp.dot(r1, wr2_ref[...], preferred_element_type=_f32)
    c = _swish(b) * rbf
    d = jnp.dot(c, wd_ref[...], preferred_element_type=_f32)
    e = _swish(d)
    xkj_ref[...] = jnp.concatenate([e, jnp.zeros_like(e)], axis=1)


def _edges(x1, rbf0, w_ji, b_ji, w_kj, b_kj, w_rbf1, w_rbf2, w_down):
    be = 2000
    grid = (E // be,)
    full = lambda shape: pl.BlockSpec(shape, lambda i: (0, 0))
    return pl.pallas_call(
        _edges_body,
        grid=grid,
        in_specs=[
            pl.BlockSpec((be, H), lambda i: (i, 0)),
            pl.BlockSpec((be, NR), lambda i: (i, 0)),
            full((H, H)), full((1, H)), full((H, H)), full((1, H)),
            full((NR, 8)), full((8, H)), full((H, F)),
        ],
        out_specs=[
            pl.BlockSpec((be, H), lambda i: (i, 0)),
            pl.BlockSpec((be, 2 * F), lambda i: (i, 0)),
        ],
        out_shape=[
            jax.ShapeDtypeStruct((E, H), _f32),
            jax.ShapeDtypeStruct((E, 2 * F), _f32),
        ],
        compiler_params=pltpu.CompilerParams(
            dimension_semantics=("parallel",)),
    )(x1, rbf0, w_ji, b_ji, w_kj, b_kj, w_rbf1, w_rbf2, w_down)


def _prod_body(sbf_ref, g_ref, w1_ref, w2_ref, out_ref):
    s1 = jnp.dot(sbf_ref[...], w1_ref[...], preferred_element_type=_f32)
    s = jnp.dot(s1, w2_ref[...], preferred_element_type=_f32)
    out_ref[...] = s * g_ref[...]


def _prod(sbf2, g2, w1b, w2b):
    bt2 = 2000
    grid = (T // 2 // bt2,)
    full = lambda shape: pl.BlockSpec(shape, lambda i: (0, 0))
    return pl.pallas_call(
        _prod_body,
        grid=grid,
        in_specs=[
            pl.BlockSpec((bt2, 2 * NSR), lambda i: (i, 0)),
            pl.BlockSpec((bt2, 2 * F), lambda i: (i, 0)),
            full((2 * NSR, 2 * F)), full((2 * F, 2 * F)),
        ],
        out_specs=pl.BlockSpec((bt2, 2 * F), lambda i: (i, 0)),
        out_shape=jax.ShapeDtypeStruct((T // 2, 2 * F), _f32),
        compiler_params=pltpu.CompilerParams(
            dimension_semantics=("parallel",)),
    )(sbf2, g2, w1b, w2b)


_sc_mesh = plsc.VectorSubcoreMesh(
    core_axis_name="c", subcore_axis_name="s",
    num_cores=NCORE, num_subcores=NSUB)


@functools.partial(
    pl.kernel,
    out_type=jax.ShapeDtypeStruct((T, F), _f32),
    mesh=_sc_mesh,
    scratch_types=[
        pltpu.VMEM((GB,), jnp.int32),
        pltpu.VMEM((GB, F), _f32),
        pltpu.SemaphoreType.DMA,
    ],
    compiler_params=pltpu.CompilerParams(use_tc_tiling_on_sc=False, needs_layout_passes=False),
)
def _sc_gather(xkj_hbm, idxkj_hbm, out_hbm, idx_v, rows_v, sem):
    wid = lax.axis_index("s") * NCORE + lax.axis_index("c")
    base = wid * TW

    def blk(g, carry):
        off = base + g * GB
        pltpu.sync_copy(idxkj_hbm.at[pl.ds(off, GB)], idx_v)
        pltpu.async_copy(xkj_hbm.at[idx_v], rows_v, sem).wait()
        pltpu.sync_copy(rows_v, out_hbm.at[pl.ds(off, GB)])
        return carry

    lax.fori_loop(0, TW // GB, blk, jnp.int32(0))


@functools.partial(
    pl.kernel,
    out_type=jax.ShapeDtypeStruct((EP, F), _f32),
    mesh=_sc_mesh,
    scratch_types=[
        pltpu.VMEM((MB,), jnp.int32),
        pltpu.VMEM((NGRP, 128), jnp.int32),
        pltpu.VMEM((NGRP, 128), jnp.int32),
        pltpu.VMEM((128, F), _f32),
        pltpu.VMEM((16, F), _f32),
        pltpu.VMEM((16,), jnp.int32),
        pltpu.VMEM((16,), jnp.int32),
        pltpu.VMEM((C,), jnp.int32),
        pltpu.VMEM_SHARED((C + 128, F), _f32),
        pltpu.SemaphoreType.DMA,
    ],
    compiler_params=pltpu.CompilerParams(use_tc_tiling_on_sc=False, needs_layout_passes=False),
)
def _sc_scatter(prod_hbm, idxji_hbm, zeros_hbm, out_hbm,
                stage, tix2d, dst2d, prows, b16, tixb, dstb, cntarr, acc, sem):
    cid = lax.axis_index("c")
    sid = lax.axis_index("s")
    iot = jnp.arange(16, dtype=jnp.int32)
    zero16 = jnp.zeros((16,), jnp.int32)

    def zcnt(i, carry):
        cntarr[pl.ds(i * 16, 16)] = zero16
        return carry
    lax.fori_loop(0, C // 16, zcnt, jnp.int32(0))

    occ0, _ = plsc.scan_count(zero16)
    occ_base = jnp.sum(jnp.where(iot == 0, occ0, 0))
    occ_off = 1 - occ_base

    def fire_b(tixv, dstv, valid):
        occ2, _ = plsc.scan_count(jnp.where(valid, dstv, -1), mask=valid)
        occ2 = occ2 + occ_off
        mx = jnp.max(jnp.where(valid, occ2, 0))

        def rnd(r, carry):
            mr = valid & (occ2 == r + 1)
            tixb[...] = jnp.where(mr, tixv, iot)
            dstb[...] = jnp.where(mr, dstv, C + iot)
            pltpu.async_copy(prod_hbm.at[tixb], b16, sem).wait()
            pltpu.sync_copy(b16, acc.at[dstb], add=True)
            return carry

        lax.fori_loop(0, mx, rnd, jnp.int32(0))

    def process_b(nb):
        def bvreg(k, carry):
            row = AROWS + lax.shift_right_logical(k, 3)
            cb = lax.bitwise_and(k, 7) * 16
            dstv = dst2d[row, pl.ds(cb, 16)]
            tixv = tix2d[row, pl.ds(cb, 16)]
            valid = (k * 16 + iot) < nb
            fire_b(tixv, dstv, valid)
            return carry

        nv = lax.shift_right_logical(nb + 15, 4)
        lax.fori_loop(0, nv, bvreg, jnp.int32(0))

    def do_pass(g, carry):
        lo = (KCH * cid + g) * C
        pltpu.sync_copy(zeros_hbm.at[pl.ds(0, ZROWS)],
                        acc.at[pl.ds(sid * ZROWS, ZROWS)])
        plsc.subcore_barrier()

        def macro(mb, carry_mb):
            base = sid * TS + mb * MB
            pltpu.sync_copy(idxji_hbm.at[pl.ds(base, MB)], stage)

            def scan16(i, cnt):
                na, nb = cnt
                v = stage[pl.ds(i * 16, 16)]
                m = (v >= lo) & (v < lo + C)
                dst = jnp.where(m, v - lo, 0)
                occ, lastm = plsc.scan_count(jnp.where(m, dst, -1), mask=m)
                prev = plsc.load_gather(cntarr, [dst])
                tocc = prev + occ + occ_off
                plsc.store_scatter(cntarr, [dst], tocc, mask=lastm & m)
                tv = base + i * 16 + iot
                ma = m & (tocc == 1)
                slot = na + plsc.cumsum(jnp.where(ma, 1, 0)) - 1
                row = lax.shift_right_logical(slot, 7)
                col = lax.bitwise_and(slot, 127)
                plsc.store_scatter(dst2d, [row, col], dst, mask=ma)
                plsc.store_scatter(tix2d, [row, col], tv, mask=ma)
                mb2 = m & (tocc > 1)
                slotb = nb + plsc.cumsum(jnp.where(mb2, 1, 0)) - 1
                rowb = AROWS + lax.shift_right_logical(slotb, 7)
                colb = lax.bitwise_and(slotb, 127)
                plsc.store_scatter(dst2d, [rowb, colb], dst, mask=mb2)
                plsc.store_scatter(tix2d, [rowb, colb], tv, mask=mb2)
                na = na + jnp.sum(jnp.where(ma, 1, 0))
                nb = nb + jnp.sum(jnp.where(mb2, 1, 0))

                def flush(nbf):
                    process_b(nbf)
                    return jnp.int32(0)

                nb = lax.cond(nb > (NGRP - AROWS) * 128 - 16, flush,
                              lambda x: x, nb)
                return (na, nb)

            na, nb = lax.fori_loop(0, MB // 16, scan16,
                                   (jnp.int32(0), jnp.int32(0)))

            r_last = jnp.full((16,), lax.shift_right_logical(na, 7), jnp.int32)
            bcol = lax.bitwise_and(na, 127)
            for j in range(8):
                colv = iot + j * 16
                mt = colv >= bcol
                plsc.store_scatter(dst2d, [r_last, colv], C + colv, mask=mt)
                plsc.store_scatter(tix2d, [r_last, colv], colv, mask=mt)

            n_grp = lax.shift_right_logical(na + 127, 7)

            def grp(g2, carry2):
                pltpu.async_copy(prod_hbm.at[tix2d.at[g2]], prows, sem).wait()
                pltpu.sync_copy(prows, acc.at[dst2d.at[g2]], add=True)
                return carry2

            lax.fori_loop(0, n_grp, grp, jnp.int32(0))
            process_b(nb)

            def zrow(k, carry):
                row = lax.shift_right_logical(k, 3)
                cb = lax.bitwise_and(k, 7) * 16
                dstv = dst2d[row, pl.ds(cb, 16)]
                valid = (k * 16 + iot) < na
                plsc.store_scatter(cntarr, [jnp.where(valid, dstv, 0)],
                                   zero16, mask=valid)
                return carry

            lax.fori_loop(0, lax.shift_right_logical(na + 15, 4), zrow,
                          jnp.int32(0))
            return carry_mb

        lax.fori_loop(0, NMB, macro, jnp.int32(0))
        plsc.subcore_barrier()
        pltpu.sync_copy(acc.at[pl.ds(sid * DRAIN, DRAIN)],
                        out_hbm.at[pl.ds(lo + sid * DRAIN, DRAIN)])
        plsc.subcore_barrier()
        return carry

    lax.fori_loop(0, KCH, do_pass, jnp.int32(0))


def _final_body(agg_ref, xji_ref, wup_ref, out_ref):
    u = jnp.dot(agg_ref[...], wup_ref[...], preferred_element_type=_f32)
    out_ref[...] = xji_ref[...] + _swish(u)


def _final(agg, x_ji, w_up):
    be = 2000
    grid = (E // be,)
    return pl.pallas_call(
        _final_body,
        grid=grid,
        in_specs=[
            pl.BlockSpec((be, F), lambda i: (i, 0)),
            pl.BlockSpec((be, H), lambda i: (i, 0)),
            pl.BlockSpec((F, H), lambda i: (0, 0)),
        ],
        out_specs=pl.BlockSpec((be, H), lambda i: (i, 0)),
        out_shape=jax.ShapeDtypeStruct((E, H), _f32),
        compiler_params=pltpu.CompilerParams(
            dimension_semantics=("parallel",)),
    )(agg, x_ji, w_up)


def kernel(x1, x2, rbf0, sbf, t, idx_kj, idx_ji,
           W_rbf1, W_rbf2, W_sbf1, W_sbf2, W_kj, b_kj, W_ji, b_ji,
           W_down, W_up):
    idx_kj = idx_kj.astype(jnp.int32)
    idx_ji = idx_ji.astype(jnp.int32)
    x_ji, xkj = _edges(x1, rbf0, W_ji, b_ji.reshape(1, H), W_kj,
                       b_kj.reshape(1, H), W_rbf1, W_rbf2, W_down)
    gathered = _sc_gather(xkj.reshape(2 * E, F), idx_kj * 2)
    w1b = jnp.block([[W_sbf1, jnp.zeros((NSR, F), _f32)],
                     [jnp.zeros((NSR, F), _f32), W_sbf1]])
    w2b = jnp.block([[W_sbf2, jnp.zeros((F, F), _f32)],
                     [jnp.zeros((F, F), _f32), W_sbf2]])
    sbf2 = sbf.reshape(T // 2, 2 * NSR)
    prod2 = _prod(sbf2, gathered.reshape(T // 2, 2 * F), w1b, w2b)
    zeros = jnp.zeros((ZROWS, F), _f32)
    agg = _sc_scatter(prod2.reshape(T, F), idx_ji, zeros)
    return _final(agg, x_ji, W_up)
